# baseline (speedup 1.0000x reference)
import functools

import jax
import jax.numpy as jnp
from jax import lax
from jax.experimental import pallas as pl
from jax.experimental.pallas import tpu as pltpu
from jax.experimental.pallas import tpu_sc as plsc

_N = 512
_HID = 128
_EMB = 128
_HEADS = 2
_LSTM_H = 128
_T = 3
_E = 16384
_G = _T * 3
_NS = 16
_EPW = _E // _NS
_SLOTS = (_G + 1) // 2
_ZW = _N * _N // _NS
_UB = 128


def _sc_counts(edges):
    mesh = plsc.VectorSubcoreMesh(core_axis_name="c", subcore_axis_name="s")

    @functools.partial(
        pl.kernel,
        mesh=mesh,
        out_type=jax.ShapeDtypeStruct((_G * _N * _N,), jnp.float32),
        scratch_types=[
            pltpu.VMEM((_EPW,), jnp.int32),
            pltpu.VMEM((_EPW,), jnp.int32),
            pltpu.VMEM((8, 128), jnp.int32),
            pltpu.VMEM((128,), jnp.float32),
            pltpu.VMEM((_ZW,), jnp.float32),
        ] + [pltpu.VMEM_SHARED((_N * _N,), jnp.float32) for _ in range(_SLOTS)],
    )
    def kfn(edges_hbm, out_hbm, src_v, dst_v, idx_v, ones_v, zero_v, *cnt_sh):
        cid = lax.axis_index("c")
        sid = lax.axis_index("s")
        base = sid * _EPW
        one16 = jnp.full((16,), 1.0, jnp.float32)
        zero16 = jnp.zeros((16,), jnp.float32)
        for i in range(8):
            ones_v[pl.ds(i * 16, 16)] = one16

        def _fz(i, carry):
            zero_v[pl.ds(i * 16, 16)] = zero16
            return carry

        lax.fori_loop(0, _ZW // 16, _fz, 0)
        for slot in range(_SLOTS):
            pltpu.sync_copy(zero_v, cnt_sh[slot].at[pl.ds(sid * _ZW, _ZW)])
        plsc.subcore_barrier()
        for g in range(_G):
            @pl.when(cid == (g % 2))
            def _(g=g):
                slot = g // 2
                pltpu.sync_copy(
                    edges_hbm.at[pl.ds(g * 2 * _E + base, _EPW)], src_v)
                pltpu.sync_copy(
                    edges_hbm.at[pl.ds((g * 2 + 1) * _E + base, _EPW)], dst_v)
                for r in range(8):
                    def _fc(i, carry, r=r):
                        o = r * 128 + i * 16
                        s16 = src_v[pl.ds(o, 16)]
                        d16 = dst_v[pl.ds(o, 16)]
                        idx_v[r, pl.ds(i * 16, 16)] = d16 * _N + s16
                        return carry

                    lax.fori_loop(0, 8, _fc, 0)
                for r in range(8):
                    pltpu.sync_copy(ones_v, cnt_sh[slot].at[idx_v.at[r]],
                                    add=True)
        plsc.subcore_barrier()
        for g in range(_G):
            @pl.when(cid == (g % 2))
            def _(g=g):
                slot = g // 2
                pltpu.sync_copy(
                    cnt_sh[slot].at[pl.ds(sid * _ZW, _ZW)],
                    out_hbm.at[pl.ds(g * _N * _N + sid * _ZW, _ZW)])

    return kfn(edges)


def _cg_body(*refs):
    cnt_ref = refs[0]
    emb = refs[1:4]
    W0 = refs[4:7]
    as0 = refs[7:10]
    ad0 = refs[10:13]
    b0 = refs[13:16]
    W1 = refs[16:19]
    as1 = refs[19:22]
    ad1 = refs[22:25]
    b1 = refs[25:28]
    cw0 = refs[28:31]
    cb0 = refs[31:34]
    cw1 = refs[34:37]
    cb1 = refs[37:40]
    z_ref = refs[40]
    leaf = refs[41:50]
    t_id = pl.program_id(0)

    ii = lax.broadcasted_iota(jnp.int32, (_N, _N), 0)
    jj = lax.broadcasted_iota(jnp.int32, (_N, _N), 1)
    eye = jnp.where(ii == jj, 1.0, 0.0).astype(jnp.float32)

    def gat(x, cnt_j, W_r, as_r, ad_r, b_r, heads):
        h = jnp.dot(x.astype(jnp.bfloat16), W_r[...].astype(jnp.bfloat16),
                    preferred_element_type=jnp.float32)
        cntI = cnt_j + eye
        outs = []
        C = h.shape[1] // heads
        for hd in range(heads):
            hh = h[:, hd * C:(hd + 1) * C]
            asr = as_r[hd:hd + 1, :]
            ads = ad_r[hd:hd + 1, :]
            arow = lax.dot_general(asr, hh, (((1,), (1,)), ((), ())),
                                   precision=lax.Precision.HIGHEST,
                                   preferred_element_type=jnp.float32)
            acol = lax.dot_general(hh, ads, (((1,), (1,)), ((), ())),
                                   precision=lax.Precision.HIGHEST,
                                   preferred_element_type=jnp.float32)
            al = acol + arow
            al = jnp.where(al >= 0.0, al, 0.2 * al)
            amax = jnp.max(al, axis=1, keepdims=True)
            ex = jnp.exp(al - amax)
            wde = cntI * ex
            den = jnp.sum(wde, axis=1, keepdims=True)
            att = wde / (den + 1e-16)
            outs.append(jnp.dot(att, hh, precision=lax.Precision.HIGHEST,
                                preferred_element_type=jnp.float32))
        o = outs[0] if heads == 1 else jnp.concatenate(outs, axis=1)
        return o + b_r[...]

    def cross(x_all, cw_refs, cb_refs):
        new = []
        for i in range(3):
            j1, j2 = (i + 1) % 3, (i + 2) % 3
            w = cw_refs[i][...]
            bs = cb_refs[i][0]

            def gate(xg):
                lin = jnp.dot(xg.astype(jnp.bfloat16),
                              w.astype(jnp.bfloat16),
                              preferred_element_type=jnp.float32)
                return jax.nn.sigmoid(lin + bs)

            xc = gate(x_all[j1]) * x_all[j1] + gate(x_all[j2]) * x_all[j2]
            tv = x_all[i] + xc
            new.append(jnp.where(tv > 0.0, tv, jnp.exp(tv) - 1.0))
        return new

    xs = [emb[j][...] for j in range(3)]
    xs = [gat(xs[j], cnt_ref[0, j], W0[j], as0[j], ad0[j], b0[j], _HEADS)
          for j in range(3)]
    xs = cross(xs, cw0, cb0)
    xs = [gat(xs[j], cnt_ref[0, j], W1[j], as1[j], ad1[j], b1[j], 1)
          for j in range(3)]
    xs = cross(xs, cw1, cb1)
    for l in range(3):
        z_ref[0, l] = xs[l]
    for tt in range(_T):
        @pl.when(t_id == tt)
        def _(tt=tt):
            for l in range(3):
                leaf[l * _T + tt][...] = xs[l]


def _crossgat_tc(cnt, emb, W0, as0, ad0, b0, W1, as1, ad1, b1,
                 cw0, cb0, cw1, cb1):
    full = lambda shape: pl.BlockSpec(shape, lambda t: (0,) * len(shape))
    smem = lambda: pl.BlockSpec(memory_space=pltpu.SMEM)
    in_specs = ([pl.BlockSpec((1, 3, _N, _N), lambda t: (t, 0, 0, 0))]
                + [full((_N, _HID))] * 3
                + [full((_HID, _HEADS * _HID))] * 3
                + [full((_HEADS, _HID))] * 6
                + [full((1, _HEADS * _HID))] * 3
                + [full((_HEADS * _HID, _EMB))] * 3
                + [full((1, _EMB))] * 6
                + [full((1, _EMB))] * 3
                + [full((_HEADS * _HID, 1))] * 3 + [smem()] * 3
                + [full((_EMB, 1))] * 3 + [smem()] * 3)
    out_specs = ([pl.BlockSpec((1, 3, _N, _EMB), lambda t: (t, 0, 0, 0))]
                 + [pl.BlockSpec((_N, _EMB), lambda t: (0, 0))] * 9)
    out_shape = ([jax.ShapeDtypeStruct((_T, 3, _N, _EMB), jnp.float32)]
                 + [jax.ShapeDtypeStruct((_N, _EMB), jnp.float32)] * 9)
    return pl.pallas_call(
        _cg_body,
        grid=(_T,),
        in_specs=in_specs,
        out_specs=out_specs,
        out_shape=out_shape,
    )(cnt, *emb, *W0, *as0, *ad0, *b0, *W1, *as1, *ad1, *b1,
      *cw0, *cb0, *cw1, *cb1)


def _dec_body(z_ref, Wih_ref, Whh_ref, bih_ref, bhh_ref,
              W1_ref, b1_ref, w2_ref, b2_ref,
              o0_ref, o1_ref, o2_ref, h_scr):
    l_id = pl.program_id(0)
    u = pl.program_id(1)

    @pl.when(u == 0)
    def _():
        bias = bih_ref[...] + bhh_ref[...]
        h = jnp.zeros((_N, _LSTM_H), jnp.float32)
        c = jnp.zeros((_N, _LSTM_H), jnp.float32)
        Wih = Wih_ref[...].astype(jnp.bfloat16)
        Whh = Whh_ref[...].astype(jnp.bfloat16)
        for t in range(_T):
            xt = z_ref[t, 0].astype(jnp.bfloat16)
            g = (lax.dot_general(xt, Wih, (((1,), (1,)), ((), ())),
                                 preferred_element_type=jnp.float32)
                 + lax.dot_general(h.astype(jnp.bfloat16), Whh,
                                   (((1,), (1,)), ((), ())),
                                   preferred_element_type=jnp.float32) + bias)
            i_ = g[:, :_LSTM_H]
            f_ = g[:, _LSTM_H:2 * _LSTM_H]
            g_ = g[:, 2 * _LSTM_H:3 * _LSTM_H]
            o_ = g[:, 3 * _LSTM_H:]
            c = jax.nn.sigmoid(f_) * c + jax.nn.sigmoid(i_) * jnp.tanh(g_)
            h = jax.nn.sigmoid(o_) * jnp.tanh(c)
        h_scr[...] = h

    hh = h_scr[...].astype(jnp.bfloat16)
    hu = h_scr[pl.ds(u * _UB, _UB), :].astype(jnp.bfloat16)
    W1u = W1_ref[:, :_LSTM_H].astype(jnp.bfloat16)
    W1v = W1_ref[:, _LSTM_H:].astype(jnp.bfloat16)
    A = lax.dot_general(hu, W1u, (((1,), (1,)), ((), ())),
                        preferred_element_type=jnp.float32)
    A = A + b1_ref[...]
    Bt = lax.dot_general(W1v, hh, (((1,), (1,)), ((), ())),
                         preferred_element_type=jnp.float32)
    acc = jnp.full((_UB, _N), b2_ref[0, 0], jnp.float32)
    for kc in range(_LSTM_H // 8):
        part = None
        for k8 in range(8):
            k = kc * 8 + k8
            w2k = w2_ref[0, k].astype(jnp.bfloat16).astype(jnp.float32)
            hidden = jnp.maximum(A[:, k:k + 1] + Bt[k:k + 1, :], 0.0)
            hidden = hidden.astype(jnp.bfloat16).astype(jnp.float32)
            term = w2k * hidden
            part = term if part is None else part + term
        acc = acc + part
    out = [o0_ref, o1_ref, o2_ref]
    for ll in range(3):
        @pl.when(l_id == ll)
        def _(ll=ll):
            out[ll][0] = acc


def _dec_tc(z, Wih, Whh, bih, bhh, W1, b1, w2, b2):
    full = lambda shape: pl.BlockSpec(shape, lambda l, u: (0,) * len(shape))
    smem = lambda: pl.BlockSpec(memory_space=pltpu.SMEM)

    def osp(ll):
        return pl.BlockSpec(
            (1, _UB, _N),
            lambda l, u, ll=ll: (
                0,
                jnp.where(l == ll, u,
                          jnp.where(l < ll, 0, _N // _UB - 1)),
                0))

    return pl.pallas_call(
        _dec_body,
        grid=(3, _N // _UB),
        in_specs=[
            pl.BlockSpec((_T, 1, _N, _EMB), lambda l, u: (0, l, 0, 0)),
            full((4 * _LSTM_H, _EMB)), full((4 * _LSTM_H, _LSTM_H)),
            full((1, 4 * _LSTM_H)), full((1, 4 * _LSTM_H)),
            full((_LSTM_H, 2 * _LSTM_H)), full((1, _LSTM_H)),
            smem(), smem(),
        ],
        out_specs=[osp(0), osp(1), osp(2)],
        out_shape=[jax.ShapeDtypeStruct((1, _N, _N), jnp.float32)] * 3,
        scratch_shapes=[pltpu.VMEM((_N, _LSTM_H), jnp.float32)],
    )(z, Wih, Whh, bih, bhh, W1, b1, w2, b2)


def kernel(edge_index_seq_list, params):
    edges = edge_index_seq_list.reshape(-1)
    cnt = _sc_counts(edges).reshape(_T, 3, _N, _N)

    gat0, gat1 = params['gat']
    cross0, cross1 = params['cross']
    emb = list(params['emb'])
    W0 = [p['W'] for p in gat0]
    as0 = [p['a_src'] for p in gat0]
    ad0 = [p['a_dst'] for p in gat0]
    b0 = [p['b'][None, :] for p in gat0]
    W1 = [p['W'] for p in gat1]
    as1 = [p['a_src'] for p in gat1]
    ad1 = [p['a_dst'] for p in gat1]
    b1 = [p['b'][None, :] for p in gat1]
    cw0 = [p['w'] for p in cross0]
    cb0 = [p['b'] for p in cross0]
    cw1 = [p['w'] for p in cross1]
    cb1 = [p['b'] for p in cross1]

    cg = _crossgat_tc(cnt, emb, W0, as0, ad0, b0, W1, as1, ad1, b1,
                      cw0, cb0, cw1, cb1)
    z = cg[0]
    leaves = cg[1:]

    lp = params['lstm']
    d = params['dec']
    preds = _dec_tc(z, lp['Wih'], lp['Whh'], lp['bih'][None, :],
                    lp['bhh'][None, :], d['W1'], d['b1'][None, :],
                    d['W2'], d['b2'][None, :])

    return (tuple(preds[l][0] for l in range(3)),
            tuple(tuple(leaves[l * _T + t] for t in range(_T))
                  for l in range(3)))


# speedup vs baseline: 1.2358x; 1.2358x over previous
import functools

import jax
import jax.numpy as jnp
from jax import lax
from jax.experimental import pallas as pl
from jax.experimental.pallas import tpu as pltpu
from jax.experimental.pallas import tpu_sc as plsc

_N = 512
_HID = 128
_EMB = 128
_HEADS = 2
_LSTM_H = 128
_T = 3
_E = 16384
_G = _T * 3
_NS = 16
_EPW = _E // _NS
_SLOTS = (_G + 1) // 2
_ZW = _N * _N // _NS
_UB = 128


def _sc_counts(edges):
    mesh = plsc.VectorSubcoreMesh(core_axis_name="c", subcore_axis_name="s")

    @functools.partial(
        pl.kernel,
        mesh=mesh,
        out_type=jax.ShapeDtypeStruct((_G * _N * _N,), jnp.float32),
        scratch_types=[
            pltpu.VMEM((_EPW,), jnp.int32),
            pltpu.VMEM((_EPW,), jnp.int32),
            pltpu.VMEM((8, 128), jnp.int32),
            pltpu.VMEM((128,), jnp.float32),
            pltpu.VMEM((_ZW,), jnp.float32),
        ] + [pltpu.VMEM_SHARED((_N * _N,), jnp.float32) for _ in range(_SLOTS)],
    )
    def kfn(edges_hbm, out_hbm, src_v, dst_v, idx_v, ones_v, zero_v, *cnt_sh):
        cid = lax.axis_index("c")
        sid = lax.axis_index("s")
        base = sid * _EPW
        one16 = jnp.full((16,), 1.0, jnp.float32)
        zero16 = jnp.zeros((16,), jnp.float32)
        for i in range(8):
            ones_v[pl.ds(i * 16, 16)] = one16

        def _fz(i, carry):
            zero_v[pl.ds(i * 16, 16)] = zero16
            return carry

        lax.fori_loop(0, _ZW // 16, _fz, 0)
        for slot in range(_SLOTS):
            pltpu.sync_copy(zero_v, cnt_sh[slot].at[pl.ds(sid * _ZW, _ZW)])
        plsc.subcore_barrier()
        for g in range(_G):
            @pl.when(cid == (g % 2))
            def _(g=g):
                slot = g // 2
                pltpu.sync_copy(
                    edges_hbm.at[pl.ds(g * 2 * _E + base, _EPW)], src_v)
                pltpu.sync_copy(
                    edges_hbm.at[pl.ds((g * 2 + 1) * _E + base, _EPW)], dst_v)
                for r in range(8):
                    def _fc(i, carry, r=r):
                        o = r * 128 + i * 16
                        s16 = src_v[pl.ds(o, 16)]
                        d16 = dst_v[pl.ds(o, 16)]
                        idx_v[r, pl.ds(i * 16, 16)] = d16 * _N + s16
                        return carry

                    lax.fori_loop(0, 8, _fc, 0)
                for r in range(8):
                    pltpu.sync_copy(ones_v, cnt_sh[slot].at[idx_v.at[r]],
                                    add=True)
        plsc.subcore_barrier()
        for g in range(_G):
            @pl.when(cid == (g % 2))
            def _(g=g):
                slot = g // 2
                pltpu.sync_copy(
                    cnt_sh[slot].at[pl.ds(sid * _ZW, _ZW)],
                    out_hbm.at[pl.ds(g * _N * _N + sid * _ZW, _ZW)])

    return kfn(edges)


def _cg_body(*refs):
    cnt_ref = refs[0]
    emb = refs[1:4]
    W0 = refs[4:7]
    as0 = refs[7:10]
    ad0 = refs[10:13]
    b0 = refs[13:16]
    W1 = refs[16:19]
    as1 = refs[19:22]
    ad1 = refs[22:25]
    b1 = refs[25:28]
    cw0 = refs[28:31]
    cb0 = refs[31:34]
    cw1 = refs[34:37]
    cb1 = refs[37:40]
    z_ref = refs[40]
    leaf = refs[41:50]
    t_id = pl.program_id(0)

    ii = lax.broadcasted_iota(jnp.int32, (_N, _N), 0)
    jj = lax.broadcasted_iota(jnp.int32, (_N, _N), 1)
    eye = jnp.where(ii == jj, 1.0, 0.0).astype(jnp.float32)

    def gat(x, cnt_j, W_r, as_r, ad_r, b_r, heads):
        h = jnp.dot(x.astype(jnp.bfloat16), W_r[...].astype(jnp.bfloat16),
                    preferred_element_type=jnp.float32)
        cntI = cnt_j + eye
        outs = []
        C = h.shape[1] // heads
        for hd in range(heads):
            hh = h[:, hd * C:(hd + 1) * C]
            asr = as_r[hd:hd + 1, :]
            ads = ad_r[hd:hd + 1, :]
            arow = lax.dot_general(asr, hh, (((1,), (1,)), ((), ())),
                                   precision=lax.Precision.HIGHEST,
                                   preferred_element_type=jnp.float32)
            acol = lax.dot_general(hh, ads, (((1,), (1,)), ((), ())),
                                   precision=lax.Precision.HIGHEST,
                                   preferred_element_type=jnp.float32)
            al = acol + arow
            al = jnp.where(al >= 0.0, al, 0.2 * al)
            amax = jnp.max(al, axis=1, keepdims=True)
            ex = jnp.exp(al - amax)
            wde = cntI * ex
            den = jnp.sum(wde, axis=1, keepdims=True)
            att = wde / (den + 1e-16)
            outs.append(jnp.dot(att, hh, precision=lax.Precision.HIGHEST,
                                preferred_element_type=jnp.float32))
        o = outs[0] if heads == 1 else jnp.concatenate(outs, axis=1)
        return o + b_r[...]

    def cross(x_all, cw_refs, cb_refs):
        new = []
        for i in range(3):
            j1, j2 = (i + 1) % 3, (i + 2) % 3
            w = cw_refs[i][...]
            bs = cb_refs[i][0]

            def gate(xg):
                lin = jnp.dot(xg.astype(jnp.bfloat16),
                              w.astype(jnp.bfloat16),
                              preferred_element_type=jnp.float32)
                return jax.nn.sigmoid(lin + bs)

            xc = gate(x_all[j1]) * x_all[j1] + gate(x_all[j2]) * x_all[j2]
            tv = x_all[i] + xc
            new.append(jnp.where(tv > 0.0, tv, jnp.exp(tv) - 1.0))
        return new

    xs = [emb[j][...] for j in range(3)]
    xs = [gat(xs[j], cnt_ref[0, j], W0[j], as0[j], ad0[j], b0[j], _HEADS)
          for j in range(3)]
    xs = cross(xs, cw0, cb0)
    xs = [gat(xs[j], cnt_ref[0, j], W1[j], as1[j], ad1[j], b1[j], 1)
          for j in range(3)]
    xs = cross(xs, cw1, cb1)
    for l in range(3):
        z_ref[0, l] = xs[l]
    for tt in range(_T):
        @pl.when(t_id == tt)
        def _(tt=tt):
            for l in range(3):
                leaf[l * _T + tt][...] = xs[l]


def _crossgat_tc(cnt, emb, W0, as0, ad0, b0, W1, as1, ad1, b1,
                 cw0, cb0, cw1, cb1):
    full = lambda shape: pl.BlockSpec(shape, lambda t: (0,) * len(shape))
    smem = lambda: pl.BlockSpec(memory_space=pltpu.SMEM)
    in_specs = ([pl.BlockSpec((1, 3, _N, _N), lambda t: (t, 0, 0, 0))]
                + [full((_N, _HID))] * 3
                + [full((_HID, _HEADS * _HID))] * 3
                + [full((_HEADS, _HID))] * 6
                + [full((1, _HEADS * _HID))] * 3
                + [full((_HEADS * _HID, _EMB))] * 3
                + [full((1, _EMB))] * 6
                + [full((1, _EMB))] * 3
                + [full((_HEADS * _HID, 1))] * 3 + [smem()] * 3
                + [full((_EMB, 1))] * 3 + [smem()] * 3)
    out_specs = ([pl.BlockSpec((1, 3, _N, _EMB), lambda t: (t, 0, 0, 0))]
                 + [pl.BlockSpec((_N, _EMB), lambda t: (0, 0))] * 9)
    out_shape = ([jax.ShapeDtypeStruct((_T, 3, _N, _EMB), jnp.float32)]
                 + [jax.ShapeDtypeStruct((_N, _EMB), jnp.float32)] * 9)
    return pl.pallas_call(
        _cg_body,
        grid=(_T,),
        in_specs=in_specs,
        out_specs=out_specs,
        out_shape=out_shape,
    )(cnt, *emb, *W0, *as0, *ad0, *b0, *W1, *as1, *ad1, *b1,
      *cw0, *cb0, *cw1, *cb1)


def _dec_body(z_ref, Wih_ref, Whh_ref, bih_ref, bhh_ref,
              W1_ref, b1c_ref, w2_ref, b2_ref,
              o0_ref, o1_ref, o2_ref, h_scr):
    l_id = pl.program_id(0)
    u = pl.program_id(1)

    @pl.when(u == 0)
    def _():
        bias = bih_ref[...] + bhh_ref[...]
        h = jnp.zeros((_N, _LSTM_H), jnp.float32)
        c = jnp.zeros((_N, _LSTM_H), jnp.float32)
        Wih = Wih_ref[...].astype(jnp.bfloat16)
        Whh = Whh_ref[...].astype(jnp.bfloat16)
        for t in range(_T):
            xt = z_ref[t, 0].astype(jnp.bfloat16)
            g = (lax.dot_general(xt, Wih, (((1,), (1,)), ((), ())),
                                 preferred_element_type=jnp.float32)
                 + lax.dot_general(h.astype(jnp.bfloat16), Whh,
                                   (((1,), (1,)), ((), ())),
                                   preferred_element_type=jnp.float32) + bias)
            i_ = g[:, :_LSTM_H]
            f_ = g[:, _LSTM_H:2 * _LSTM_H]
            g_ = g[:, 2 * _LSTM_H:3 * _LSTM_H]
            o_ = g[:, 3 * _LSTM_H:]
            c = jax.nn.sigmoid(f_) * c + jax.nn.sigmoid(i_) * jnp.tanh(g_)
            h = jax.nn.sigmoid(o_) * jnp.tanh(c)
        h_scr[...] = h

    hh = h_scr[...].astype(jnp.bfloat16)
    hu = h_scr[pl.ds(u * _UB, _UB), :].astype(jnp.bfloat16)
    W1u = W1_ref[:, :_LSTM_H].astype(jnp.bfloat16)
    W1v = W1_ref[:, _LSTM_H:].astype(jnp.bfloat16)
    At = lax.dot_general(W1u, hu, (((1,), (1,)), ((), ())),
                         preferred_element_type=jnp.float32)
    Bt = lax.dot_general(W1v, hh, (((1,), (1,)), ((), ())),
                         preferred_element_type=jnp.float32)
    At = At + b1c_ref[...]
    w2cat = jnp.concatenate([w2_ref[...]] * 8, axis=1)
    w2bc = jnp.broadcast_to(w2cat, (8, 8 * _LSTM_H))
    rowi = lax.broadcasted_iota(jnp.int32, (8, 8 * _LSTM_H), 0)
    coli = lax.broadcasted_iota(jnp.int32, (8, 8 * _LSTM_H), 1)
    W2bd = jnp.where(coli // _LSTM_H == rowi, w2bc,
                     0.0).astype(jnp.bfloat16)
    b2s = b2_ref[0, 0]
    rows = []
    for ub in range(_UB // 8):
        pieces = []
        for r in range(8):
            uu = ub * 8 + r
            hid = jnp.maximum(At[:, uu:uu + 1] + Bt, 0.0)
            pieces.append(hid.astype(jnp.bfloat16))
        r8 = jnp.concatenate(pieces, axis=0)
        out8 = lax.dot_general(W2bd, r8, (((1,), (0,)), ((), ())),
                               preferred_element_type=jnp.float32)
        rows.append(out8 + b2s)
    acc = jnp.concatenate(rows, axis=0)
    out = [o0_ref, o1_ref, o2_ref]
    for ll in range(3):
        @pl.when(l_id == ll)
        def _(ll=ll):
            out[ll][0] = acc


def _dec_tc(z, Wih, Whh, bih, bhh, W1, b1c, w2, b2):
    full = lambda shape: pl.BlockSpec(shape, lambda l, u: (0,) * len(shape))
    smem = lambda: pl.BlockSpec(memory_space=pltpu.SMEM)

    def osp(ll):
        return pl.BlockSpec(
            (1, _UB, _N),
            lambda l, u, ll=ll: (
                0,
                jnp.where(l == ll, u,
                          jnp.where(l < ll, 0, _N // _UB - 1)),
                0))

    return pl.pallas_call(
        _dec_body,
        grid=(3, _N // _UB),
        in_specs=[
            pl.BlockSpec((_T, 1, _N, _EMB), lambda l, u: (0, l, 0, 0)),
            full((4 * _LSTM_H, _EMB)), full((4 * _LSTM_H, _LSTM_H)),
            full((1, 4 * _LSTM_H)), full((1, 4 * _LSTM_H)),
            full((_LSTM_H, 2 * _LSTM_H)), full((_LSTM_H, 1)),
            full((1, _LSTM_H)), smem(),
        ],
        out_specs=[osp(0), osp(1), osp(2)],
        out_shape=[jax.ShapeDtypeStruct((1, _N, _N), jnp.float32)] * 3,
        scratch_shapes=[pltpu.VMEM((_N, _LSTM_H), jnp.float32)],
    )(z, Wih, Whh, bih, bhh, W1, b1c, w2, b2)


def kernel(edge_index_seq_list, params):
    edges = edge_index_seq_list.reshape(-1)
    cnt = _sc_counts(edges).reshape(_T, 3, _N, _N)

    gat0, gat1 = params['gat']
    cross0, cross1 = params['cross']
    emb = list(params['emb'])
    W0 = [p['W'] for p in gat0]
    as0 = [p['a_src'] for p in gat0]
    ad0 = [p['a_dst'] for p in gat0]
    b0 = [p['b'][None, :] for p in gat0]
    W1 = [p['W'] for p in gat1]
    as1 = [p['a_src'] for p in gat1]
    ad1 = [p['a_dst'] for p in gat1]
    b1 = [p['b'][None, :] for p in gat1]
    cw0 = [p['w'] for p in cross0]
    cb0 = [p['b'] for p in cross0]
    cw1 = [p['w'] for p in cross1]
    cb1 = [p['b'] for p in cross1]

    cg = _crossgat_tc(cnt, emb, W0, as0, ad0, b0, W1, as1, ad1, b1,
                      cw0, cb0, cw1, cb1)
    z = cg[0]
    leaves = cg[1:]

    lp = params['lstm']
    d = params['dec']
    preds = _dec_tc(z, lp['Wih'], lp['Whh'], lp['bih'][None, :],
                    lp['bhh'][None, :], d['W1'], d['b1'][:, None],
                    d['W2'], d['b2'][None, :])

    return (tuple(preds[l][0] for l in range(3)),
            tuple(tuple(leaves[l * _T + t] for t in range(_T))
                  for l in range(3)))


# speedup vs baseline: 1.3397x; 1.0841x over previous
import functools

import jax
import jax.numpy as jnp
from jax import lax
from jax.experimental import pallas as pl
from jax.experimental.pallas import tpu as pltpu
from jax.experimental.pallas import tpu_sc as plsc

_N = 512
_HID = 128
_EMB = 128
_HEADS = 2
_LSTM_H = 128
_T = 3
_E = 16384
_G = _T * 3
_NS = 16
_EPW = _E // _NS
_SLOTS = (_G + 1) // 2
_ZW = _N * _N // _NS
_UB = 128


def _sc_counts(edges):
    mesh = plsc.VectorSubcoreMesh(core_axis_name="c", subcore_axis_name="s")

    @functools.partial(
        pl.kernel,
        mesh=mesh,
        out_type=jax.ShapeDtypeStruct((_G * _N * _N,), jnp.float32),
        scratch_types=[
            pltpu.VMEM((_EPW,), jnp.int32),
            pltpu.VMEM((_EPW,), jnp.int32),
            pltpu.VMEM((8, 128), jnp.int32),
            pltpu.VMEM((128,), jnp.float32),
            pltpu.VMEM((_ZW,), jnp.float32),
        ] + [pltpu.VMEM_SHARED((_N * _N,), jnp.float32) for _ in range(_SLOTS)],
    )
    def kfn(edges_hbm, out_hbm, src_v, dst_v, idx_v, ones_v, zero_v, *cnt_sh):
        cid = lax.axis_index("c")
        sid = lax.axis_index("s")
        base = sid * _EPW
        one16 = jnp.full((16,), 1.0, jnp.float32)
        zero16 = jnp.zeros((16,), jnp.float32)
        for i in range(8):
            ones_v[pl.ds(i * 16, 16)] = one16

        def _fz(i, carry):
            zero_v[pl.ds(i * 16, 16)] = zero16
            return carry

        lax.fori_loop(0, _ZW // 16, _fz, 0)
        for slot in range(_SLOTS):
            pltpu.sync_copy(zero_v, cnt_sh[slot].at[pl.ds(sid * _ZW, _ZW)])
        plsc.subcore_barrier()
        for g in range(_G):
            @pl.when(cid == (g % 2))
            def _(g=g):
                slot = g // 2
                pltpu.sync_copy(
                    edges_hbm.at[pl.ds(g * 2 * _E + base, _EPW)], src_v)
                pltpu.sync_copy(
                    edges_hbm.at[pl.ds((g * 2 + 1) * _E + base, _EPW)], dst_v)
                for r in range(8):
                    def _fc(i, carry, r=r):
                        o = r * 128 + i * 16
                        s16 = src_v[pl.ds(o, 16)]
                        d16 = dst_v[pl.ds(o, 16)]
                        idx_v[r, pl.ds(i * 16, 16)] = d16 * _N + s16
                        return carry

                    lax.fori_loop(0, 8, _fc, 0)
                for r in range(8):
                    pltpu.sync_copy(ones_v, cnt_sh[slot].at[idx_v.at[r]],
                                    add=True)
        plsc.subcore_barrier()
        for g in range(_G):
            @pl.when(cid == (g % 2))
            def _(g=g):
                slot = g // 2
                pltpu.sync_copy(
                    cnt_sh[slot].at[pl.ds(sid * _ZW, _ZW)],
                    out_hbm.at[pl.ds(g * _N * _N + sid * _ZW, _ZW)])

    return kfn(edges)


def _cg_body(*refs):
    cnt_ref = refs[0]
    emb = refs[1:4]
    W0 = refs[4:7]
    as0 = refs[7:10]
    ad0 = refs[10:13]
    b0 = refs[13:16]
    W1 = refs[16:19]
    as1 = refs[19:22]
    ad1 = refs[22:25]
    b1 = refs[25:28]
    cw0 = refs[28:31]
    cb0 = refs[31:34]
    cw1 = refs[34:37]
    cb1 = refs[37:40]
    z_ref = refs[40]
    leaf = refs[41:50]
    t_id = pl.program_id(0)

    ii = lax.broadcasted_iota(jnp.int32, (_N, _N), 0)
    jj = lax.broadcasted_iota(jnp.int32, (_N, _N), 1)
    eye = jnp.where(ii == jj, 1.0, 0.0).astype(jnp.float32)

    def gat(x, cnt_j, W_r, as_r, ad_r, b_r, heads):
        h = jnp.dot(x.astype(jnp.bfloat16), W_r[...].astype(jnp.bfloat16),
                    preferred_element_type=jnp.float32)
        cntI = cnt_j + eye
        outs = []
        C = h.shape[1] // heads
        for hd in range(heads):
            hh = h[:, hd * C:(hd + 1) * C]
            asr = as_r[hd:hd + 1, :]
            ads = ad_r[hd:hd + 1, :]
            arow = lax.dot_general(asr, hh, (((1,), (1,)), ((), ())),
                                   precision=lax.Precision.HIGHEST,
                                   preferred_element_type=jnp.float32)
            acol = lax.dot_general(hh, ads, (((1,), (1,)), ((), ())),
                                   precision=lax.Precision.HIGHEST,
                                   preferred_element_type=jnp.float32)
            al = acol + arow
            al = jnp.where(al >= 0.0, al, 0.2 * al)
            amax = jnp.max(al, axis=1, keepdims=True)
            ex = jnp.exp(al - amax)
            wde = cntI * ex
            den = jnp.sum(wde, axis=1, keepdims=True)
            att = wde / (den + 1e-16)
            a16 = att.astype(jnp.bfloat16)
            ar16 = (att - a16.astype(jnp.float32)).astype(jnp.bfloat16)
            h16 = hh.astype(jnp.bfloat16)
            hr16 = (hh - h16.astype(jnp.float32)).astype(jnp.bfloat16)
            mm = lambda p, q: jnp.dot(p, q,
                                      preferred_element_type=jnp.float32)
            outs.append(mm(a16, h16) + (mm(ar16, h16) + mm(a16, hr16)))
        o = outs[0] if heads == 1 else jnp.concatenate(outs, axis=1)
        return o + b_r[...]

    def cross(x_all, cw_refs, cb_refs):
        new = []
        for i in range(3):
            j1, j2 = (i + 1) % 3, (i + 2) % 3
            w = cw_refs[i][...]
            bs = cb_refs[i][0]

            def gate(xg):
                lin = jnp.dot(xg.astype(jnp.bfloat16),
                              w.astype(jnp.bfloat16),
                              preferred_element_type=jnp.float32)
                return jax.nn.sigmoid(lin + bs)

            xc = gate(x_all[j1]) * x_all[j1] + gate(x_all[j2]) * x_all[j2]
            tv = x_all[i] + xc
            new.append(jnp.where(tv > 0.0, tv, jnp.exp(tv) - 1.0))
        return new

    xs = [emb[j][...] for j in range(3)]
    xs = [gat(xs[j], cnt_ref[0, j], W0[j], as0[j], ad0[j], b0[j], _HEADS)
          for j in range(3)]
    xs = cross(xs, cw0, cb0)
    xs = [gat(xs[j], cnt_ref[0, j], W1[j], as1[j], ad1[j], b1[j], 1)
          for j in range(3)]
    xs = cross(xs, cw1, cb1)
    for l in range(3):
        z_ref[0, l] = xs[l]
    for tt in range(_T):
        @pl.when(t_id == tt)
        def _(tt=tt):
            for l in range(3):
                leaf[l * _T + tt][...] = xs[l]


def _crossgat_tc(cnt, emb, W0, as0, ad0, b0, W1, as1, ad1, b1,
                 cw0, cb0, cw1, cb1):
    full = lambda shape: pl.BlockSpec(shape, lambda t: (0,) * len(shape))
    smem = lambda: pl.BlockSpec(memory_space=pltpu.SMEM)
    in_specs = ([pl.BlockSpec((1, 3, _N, _N), lambda t: (t, 0, 0, 0))]
                + [full((_N, _HID))] * 3
                + [full((_HID, _HEADS * _HID))] * 3
                + [full((_HEADS, _HID))] * 6
                + [full((1, _HEADS * _HID))] * 3
                + [full((_HEADS * _HID, _EMB))] * 3
                + [full((1, _EMB))] * 6
                + [full((1, _EMB))] * 3
                + [full((_HEADS * _HID, 1))] * 3 + [smem()] * 3
                + [full((_EMB, 1))] * 3 + [smem()] * 3)
    out_specs = ([pl.BlockSpec((1, 3, _N, _EMB), lambda t: (t, 0, 0, 0))]
                 + [pl.BlockSpec((_N, _EMB), lambda t: (0, 0))] * 9)
    out_shape = ([jax.ShapeDtypeStruct((_T, 3, _N, _EMB), jnp.float32)]
                 + [jax.ShapeDtypeStruct((_N, _EMB), jnp.float32)] * 9)
    return pl.pallas_call(
        _cg_body,
        grid=(_T,),
        in_specs=in_specs,
        out_specs=out_specs,
        out_shape=out_shape,
    )(cnt, *emb, *W0, *as0, *ad0, *b0, *W1, *as1, *ad1, *b1,
      *cw0, *cb0, *cw1, *cb1)


def _dec_body(z_ref, Wih_ref, Whh_ref, bih_ref, bhh_ref,
              W1_ref, b1c_ref, w2_ref, b2_ref,
              o0_ref, o1_ref, o2_ref, h_scr):
    l_id = pl.program_id(0)
    u = pl.program_id(1)

    @pl.when(u == 0)
    def _():
        bias = bih_ref[...] + bhh_ref[...]
        h = jnp.zeros((_N, _LSTM_H), jnp.float32)
        c = jnp.zeros((_N, _LSTM_H), jnp.float32)
        Wih = Wih_ref[...].astype(jnp.bfloat16)
        Whh = Whh_ref[...].astype(jnp.bfloat16)
        for t in range(_T):
            xt = z_ref[t, 0].astype(jnp.bfloat16)
            g = (lax.dot_general(xt, Wih, (((1,), (1,)), ((), ())),
                                 preferred_element_type=jnp.float32)
                 + lax.dot_general(h.astype(jnp.bfloat16), Whh,
                                   (((1,), (1,)), ((), ())),
                                   preferred_element_type=jnp.float32) + bias)
            i_ = g[:, :_LSTM_H]
            f_ = g[:, _LSTM_H:2 * _LSTM_H]
            g_ = g[:, 2 * _LSTM_H:3 * _LSTM_H]
            o_ = g[:, 3 * _LSTM_H:]
            c = jax.nn.sigmoid(f_) * c + jax.nn.sigmoid(i_) * jnp.tanh(g_)
            h = jax.nn.sigmoid(o_) * jnp.tanh(c)
        h_scr[...] = h

    hh = h_scr[...].astype(jnp.bfloat16)
    hu = h_scr[pl.ds(u * _UB, _UB), :].astype(jnp.bfloat16)
    W1u = W1_ref[:, :_LSTM_H].astype(jnp.bfloat16)
    W1v = W1_ref[:, _LSTM_H:].astype(jnp.bfloat16)
    At = lax.dot_general(W1u, hu, (((1,), (1,)), ((), ())),
                         preferred_element_type=jnp.float32)
    Bt = lax.dot_general(W1v, hh, (((1,), (1,)), ((), ())),
                         preferred_element_type=jnp.float32)
    At = At + b1c_ref[...]
    w2cat = jnp.concatenate([w2_ref[...]] * 8, axis=1)
    w2bc = jnp.broadcast_to(w2cat, (8, 8 * _LSTM_H))
    rowi = lax.broadcasted_iota(jnp.int32, (8, 8 * _LSTM_H), 0)
    coli = lax.broadcasted_iota(jnp.int32, (8, 8 * _LSTM_H), 1)
    W2bd = jnp.where(coli // _LSTM_H == rowi, w2bc,
                     0.0).astype(jnp.bfloat16)
    b2s = b2_ref[0, 0]
    rows = []
    for ub in range(_UB // 8):
        pieces = []
        for r in range(8):
            uu = ub * 8 + r
            hid = jnp.maximum(At[:, uu:uu + 1] + Bt, 0.0)
            pieces.append(hid.astype(jnp.bfloat16))
        r8 = jnp.concatenate(pieces, axis=0)
        out8 = lax.dot_general(W2bd, r8, (((1,), (0,)), ((), ())),
                               preferred_element_type=jnp.float32)
        rows.append(out8 + b2s)
    acc = jnp.concatenate(rows, axis=0)
    out = [o0_ref, o1_ref, o2_ref]
    for ll in range(3):
        @pl.when(l_id == ll)
        def _(ll=ll):
            out[ll][0] = acc


def _dec_tc(z, Wih, Whh, bih, bhh, W1, b1c, w2, b2):
    full = lambda shape: pl.BlockSpec(shape, lambda l, u: (0,) * len(shape))
    smem = lambda: pl.BlockSpec(memory_space=pltpu.SMEM)

    def osp(ll):
        return pl.BlockSpec(
            (1, _UB, _N),
            lambda l, u, ll=ll: (
                0,
                jnp.where(l == ll, u,
                          jnp.where(l < ll, 0, _N // _UB - 1)),
                0))

    return pl.pallas_call(
        _dec_body,
        grid=(3, _N // _UB),
        in_specs=[
            pl.BlockSpec((_T, 1, _N, _EMB), lambda l, u: (0, l, 0, 0)),
            full((4 * _LSTM_H, _EMB)), full((4 * _LSTM_H, _LSTM_H)),
            full((1, 4 * _LSTM_H)), full((1, 4 * _LSTM_H)),
            full((_LSTM_H, 2 * _LSTM_H)), full((_LSTM_H, 1)),
            full((1, _LSTM_H)), smem(),
        ],
        out_specs=[osp(0), osp(1), osp(2)],
        out_shape=[jax.ShapeDtypeStruct((1, _N, _N), jnp.float32)] * 3,
        scratch_shapes=[pltpu.VMEM((_N, _LSTM_H), jnp.float32)],
    )(z, Wih, Whh, bih, bhh, W1, b1c, w2, b2)


def kernel(edge_index_seq_list, params):
    edges = edge_index_seq_list.reshape(-1)
    cnt = _sc_counts(edges).reshape(_T, 3, _N, _N)

    gat0, gat1 = params['gat']
    cross0, cross1 = params['cross']
    emb = list(params['emb'])
    W0 = [p['W'] for p in gat0]
    as0 = [p['a_src'] for p in gat0]
    ad0 = [p['a_dst'] for p in gat0]
    b0 = [p['b'][None, :] for p in gat0]
    W1 = [p['W'] for p in gat1]
    as1 = [p['a_src'] for p in gat1]
    ad1 = [p['a_dst'] for p in gat1]
    b1 = [p['b'][None, :] for p in gat1]
    cw0 = [p['w'] for p in cross0]
    cb0 = [p['b'] for p in cross0]
    cw1 = [p['w'] for p in cross1]
    cb1 = [p['b'] for p in cross1]

    cg = _crossgat_tc(cnt, emb, W0, as0, ad0, b0, W1, as1, ad1, b1,
                      cw0, cb0, cw1, cb1)
    z = cg[0]
    leaves = cg[1:]

    lp = params['lstm']
    d = params['dec']
    preds = _dec_tc(z, lp['Wih'], lp['Whh'], lp['bih'][None, :],
                    lp['bhh'][None, :], d['W1'], d['b1'][:, None],
                    d['W2'], d['b2'][None, :])

    return (tuple(preds[l][0] for l in range(3)),
            tuple(tuple(leaves[l * _T + t] for t in range(_T))
                  for l in range(3)))
